# baseline (speedup 1.0000x reference)
import functools

import jax
import jax.numpy as jnp
from jax import lax
from jax.experimental import pallas as pl
from jax.experimental.pallas import tpu as pltpu
from jax.experimental.pallas import tpu_sc as plsc

NC, NS = 2, 16
NW = NC * NS
B = 16384
D = 128
NREL = 1000
CHUNK = 128
ROWS_PER_W = B // NW
NCHUNK = ROWS_PER_W // CHUNK
NARR = 4

GDEPTH = 4
NBIG = 3
NSTEP = NARR * NCHUNK

_mesh = plsc.VectorSubcoreMesh(
    core_axis_name="c", subcore_axis_name="s", num_cores=NC, num_subcores=NS
)


@functools.partial(
    pl.kernel,
    out_type=[jax.ShapeDtypeStruct((B, D), jnp.float32) for _ in range(NARR)],
    mesh=_mesh,
    scratch_types=[
        pltpu.VMEM((NARR, ROWS_PER_W), jnp.int32),
        pltpu.VMEM((NBIG, 2 * CHUNK, D), jnp.float32),
        pltpu.VMEM_SHARED((NREL, D), jnp.float32),
        [pltpu.SemaphoreType.DMA for _ in range(2 * NBIG)],
        [pltpu.SemaphoreType.DMA for _ in range(NBIG)],
        pltpu.SemaphoreType.DMA,
    ],
)
def _gather4(head_i, rel_i, tail_i, neg_i, ent_hbm, relemb_hbm,
             out_head, out_rel, out_tail, out_neg,
             idx_v, rows_v, rel_sh, gsems, wsems, isem):
    cid = lax.axis_index("c")
    sid = lax.axis_index("s")
    wid = sid * NC + cid
    base = wid * ROWS_PER_W

    rel_desc = [None, None]
    rel_off = pl.multiple_of(sid * 128, 8)
    @pl.when(sid < 7)
    def _():
        rel_desc[0] = pltpu.async_copy(
            relemb_hbm.at[pl.ds(rel_off, 128)],
            rel_sh.at[pl.ds(rel_off, 128)],
            isem,
        )
    @pl.when(sid == 7)
    def _():
        rel_desc[1] = pltpu.async_copy(
            relemb_hbm.at[pl.ds(896, NREL - 896)],
            rel_sh.at[pl.ds(896, NREL - 896)],
            isem,
        )

    idx_in = (head_i, rel_i, tail_i, neg_i)
    head_d = pltpu.async_copy(
        idx_in[0].at[pl.ds(base, ROWS_PER_W)], idx_v.at[0], gsems[2 * NBIG - 1]
    )
    descs = [
        pltpu.async_copy(
            idx_in[a].at[pl.ds(base, ROWS_PER_W)], idx_v.at[a], wsems[a - 1]
        )
        for a in range(1, NARR)
    ]
    head_d.wait()

    tables = (ent_hbm, rel_sh, ent_hbm, ent_hbm)
    outs = (out_head, out_rel, out_tail, out_neg)
    order = (0, 2, 3, 1)
    steps = [(a, c) for a in order for c in range(NCHUNK)]
    g_desc = [None] * (2 * NBIG)
    w_desc = [None] * NBIG

    def start_gather(i):
        a, c = steps[i]
        big, h = (i // 2) % NBIG, i % 2
        s = big * 2 + h
        g_desc[s] = pltpu.async_copy(
            tables[a].at[idx_v.at[a, pl.ds(c * CHUNK, CHUNK)]],
            rows_v.at[big, pl.ds(h * CHUNK, CHUNK)],
            gsems[s],
        )

    def start_write(pair):
        a, c = steps[2 * pair]
        big = pair % NBIG
        w_desc[big] = pltpu.async_copy(
            rows_v.at[big],
            outs[a].at[pl.ds(base + c * CHUNK, 2 * CHUNK)],
            wsems[big],
        )

    FIRST_REL = NSTEP - NCHUNK
    for i in range(min(GDEPTH, NCHUNK)):
        start_gather(i)
    for d_ in descs:
        d_.wait()
    for i in range(NCHUNK, GDEPTH):
        start_gather(i)
    for i in range(NSTEP):
        if i + GDEPTH == FIRST_REL:
            @pl.when(sid < 7)
            def _():
                rel_desc[0].wait()
            @pl.when(sid == 7)
            def _():
                rel_desc[1].wait()
            plsc.subcore_barrier()
        big, h = (i // 2) % NBIG, i % 2
        g_desc[big * 2 + h].wait()
        k = i + GDEPTH
        if k < NSTEP:
            if k % 2 == 0:
                kb = (k // 2) % NBIG
                if w_desc[kb] is not None:
                    w_desc[kb].wait()
            start_gather(k)
        if h == 1:
            start_write(i // 2)
    for pair in range(NSTEP // 2 - NBIG, NSTEP // 2):
        if w_desc[pair % NBIG] is not None:
            w_desc[pair % NBIG].wait()


def kernel(head, tail, rel, neg, entity_emb, relation_emb):
    head_e, rel_e, tail_e, neg_e = _gather4(
        head.astype(jnp.int32),
        rel.astype(jnp.int32),
        tail.astype(jnp.int32),
        neg.astype(jnp.int32),
        entity_emb,
        relation_emb,
    )
    return (head_e, rel_e, tail_e, neg_e)


# speedup vs baseline: 1.0143x; 1.0143x over previous
import functools

import jax
import jax.numpy as jnp
from jax import lax
from jax.experimental import pallas as pl
from jax.experimental.pallas import tpu as pltpu
from jax.experimental.pallas import tpu_sc as plsc

NC, NS = 2, 16
NW = NC * NS
B = 16384
D = 128
NREL = 1000
CHUNK = 128
ROWS_PER_W = B // NW
NCHUNK = ROWS_PER_W // CHUNK
NARR = 4

GDEPTH = 5
NBUF = 7
NSTEP = NARR * NCHUNK

_mesh = plsc.VectorSubcoreMesh(
    core_axis_name="c", subcore_axis_name="s", num_cores=NC, num_subcores=NS
)


@functools.partial(
    pl.kernel,
    out_type=[jax.ShapeDtypeStruct((B, D), jnp.float32) for _ in range(NARR)],
    mesh=_mesh,
    scratch_types=[
        pltpu.VMEM((NARR, ROWS_PER_W), jnp.int32),
        pltpu.VMEM((NBUF, CHUNK, D), jnp.float32),
        pltpu.VMEM_SHARED((NREL, D), jnp.float32),
        [pltpu.SemaphoreType.DMA for _ in range(NBUF)],
        [pltpu.SemaphoreType.DMA for _ in range(NBUF)],
        pltpu.SemaphoreType.DMA,
    ],
)
def _gather4(head_i, rel_i, tail_i, neg_i, ent_hbm, relemb_hbm,
             out_head, out_rel, out_tail, out_neg,
             idx_v, rows_v, rel_sh, gsems, wsems, isem):
    cid = lax.axis_index("c")
    sid = lax.axis_index("s")
    wid = sid * NC + cid
    base = wid * ROWS_PER_W

    rel_desc = [None, None]
    rel_off = pl.multiple_of(sid * 128, 8)
    @pl.when(sid < 7)
    def _():
        rel_desc[0] = pltpu.async_copy(
            relemb_hbm.at[pl.ds(rel_off, 128)],
            rel_sh.at[pl.ds(rel_off, 128)],
            isem,
        )
    @pl.when(sid == 7)
    def _():
        rel_desc[1] = pltpu.async_copy(
            relemb_hbm.at[pl.ds(896, NREL - 896)],
            rel_sh.at[pl.ds(896, NREL - 896)],
            isem,
        )

    idx_in = (head_i, rel_i, tail_i, neg_i)
    head_d = pltpu.async_copy(
        idx_in[0].at[pl.ds(base, ROWS_PER_W)], idx_v.at[0], gsems[NBUF - 1]
    )
    descs = [
        pltpu.async_copy(
            idx_in[a].at[pl.ds(base, ROWS_PER_W)], idx_v.at[a], wsems[a]
        )
        for a in range(1, NARR)
    ]
    head_d.wait()

    tables = (ent_hbm, rel_sh, ent_hbm, ent_hbm)
    outs = (out_head, out_rel, out_tail, out_neg)
    order = (0, 2, 3, 1)
    steps = [(a, c) for a in order for c in range(NCHUNK)]
    g_desc = [None] * NBUF
    w_desc = [None] * NBUF

    def start_gather(i):
        a, c = steps[i]
        b = i % NBUF
        g_desc[b] = pltpu.async_copy(
            tables[a].at[idx_v.at[a, pl.ds(c * CHUNK, CHUNK)]],
            rows_v.at[b],
            gsems[b],
        )

    def start_write(i):
        a, c = steps[i]
        b = i % NBUF
        w_desc[b] = pltpu.async_copy(
            rows_v.at[b], outs[a].at[pl.ds(base + c * CHUNK, CHUNK)], wsems[b]
        )

    FIRST_REL = NSTEP - NCHUNK
    for i in range(min(GDEPTH, NCHUNK)):
        start_gather(i)
    for d_ in descs:
        d_.wait()
    for i in range(NCHUNK, GDEPTH):
        start_gather(i)
    for i in range(NSTEP):
        if i + GDEPTH == FIRST_REL:
            @pl.when(sid < 7)
            def _():
                rel_desc[0].wait()
            @pl.when(sid == 7)
            def _():
                rel_desc[1].wait()
            plsc.subcore_barrier()
        b = i % NBUF
        g_desc[b].wait()
        if i + GDEPTH < NSTEP:
            nb = (i + GDEPTH) % NBUF
            if w_desc[nb] is not None:
                w_desc[nb].wait()
            start_gather(i + GDEPTH)
        start_write(i)
    for i in range(NSTEP - NBUF, NSTEP):
        if w_desc[i % NBUF] is not None:
            w_desc[i % NBUF].wait()


def kernel(head, tail, rel, neg, entity_emb, relation_emb):
    head_e, rel_e, tail_e, neg_e = _gather4(
        head.astype(jnp.int32),
        rel.astype(jnp.int32),
        tail.astype(jnp.int32),
        neg.astype(jnp.int32),
        entity_emb,
        relation_emb,
    )
    return (head_e, rel_e, tail_e, neg_e)
